# baseline (speedup 1.0000x reference)
import functools

import jax
import jax.numpy as jnp
from jax import lax
from jax.experimental import pallas as pl
from jax.experimental.pallas import tpu as pltpu
from jax.experimental.pallas import tpu_sc as plsc

HIDDEN = 1024
EPS = 1e-12

NC = 2
NS = 16
NW = NC * NS

CHUNK = 16
NBUF = 6
PRIME = 4


def _gather_sc(table, idx, row0, b):
    b_per_w = b // NW
    n_chunks = b_per_w // CHUNK
    n_quads = n_chunks // NBUF
    mesh = plsc.VectorSubcoreMesh(core_axis_name="c", subcore_axis_name="s")

    @functools.partial(
        pl.kernel,
        mesh=mesh,
        out_type=jax.ShapeDtypeStruct((b, HIDDEN), jnp.float32),
        scratch_types=[
            pltpu.VMEM((b_per_w,), jnp.int32),
        ] + [pltpu.VMEM((CHUNK, HIDDEN), jnp.float32)] * NBUF
          + [pltpu.SemaphoreType.DMA] * (2 * NBUF),
    )
    def k(table_hbm, idx_hbm, out_hbm, idx_v, *rest):
        bufs = rest[:NBUF]
        gs = rest[NBUF:2 * NBUF]
        ws = rest[2 * NBUF:]
        wid = lax.axis_index("s") * NC + lax.axis_index("c")
        base = wid * b_per_w
        pltpu.sync_copy(idx_hbm.at[pl.ds(row0 + base, b_per_w)], idx_v)

        def start_gather(c, buf, sem):
            pltpu.async_copy(
                table_hbm.at[idx_v.at[pl.ds(c * CHUNK, CHUNK)]], buf, sem
            )

        def wait_gather(buf, sem):
            pltpu.make_async_copy(
                table_hbm.at[idx_v.at[pl.ds(0, CHUNK)]], buf, sem
            ).wait()

        def start_write(c, buf, sem):
            pltpu.async_copy(buf, out_hbm.at[pl.ds(base + c * CHUNK, CHUNK)], sem)

        def wait_write(buf, sem):
            pltpu.make_async_copy(
                buf, out_hbm.at[pl.ds(base, CHUNK)], sem
            ).wait()

        for c0 in range(PRIME):
            start_gather(c0, bufs[c0 % NBUF], gs[c0 % NBUF])

        for c in range(n_chunks):
            bix = c % NBUF
            wait_gather(bufs[bix], gs[bix])
            start_write(c, bufs[bix], ws[bix])
            nxt = c + PRIME
            if nxt < n_chunks:
                pb = nxt % NBUF
                if nxt >= NBUF:
                    wait_write(bufs[pb], ws[pb])
                start_gather(nxt, bufs[pb], gs[pb])

        for bix in range(NBUF):
            if bix < n_chunks:
                wait_write(bufs[bix], ws[bix])

    return k(table, idx)


BT = 2048


def _layernorm_tc_slice(x, gamma, beta, acc, row0, b_total, nrows=None):
    sl = nrows if nrows is not None else x.shape[0]
    off = row0 // BT

    def body(*refs):
        x_ref, g_ref, b_ref, o_ref = refs[-4:]
        v = x_ref[...]
        m = jnp.mean(v, axis=1, keepdims=True)
        c = v - m
        var = jnp.mean(c * c, axis=1, keepdims=True)
        o_ref[...] = c * lax.rsqrt(var + EPS) * g_ref[...] + b_ref[...]

    x_spec = pl.BlockSpec((BT, HIDDEN), lambda i: (i, 0))
    vec_spec = pl.BlockSpec((1, HIDDEN), lambda i: (0, 0))
    if acc is None:
        in_specs = [x_spec, vec_spec, vec_spec]
        args = (x, gamma, beta)
        aliases = {}
    else:
        in_specs = [pl.BlockSpec(memory_space=pl.ANY), x_spec, vec_spec,
                    vec_spec]
        args = (acc, x, gamma, beta)
        aliases = {0: 0}
    return pl.pallas_call(
        body,
        grid=(sl // BT,),
        in_specs=in_specs,
        out_specs=pl.BlockSpec((BT, HIDDEN), lambda i: (off + i, 0)),
        out_shape=jax.ShapeDtypeStruct((b_total, HIDDEN), jnp.float32),
        input_output_aliases=aliases,
    )(*args)


SLICES = (4096, 10240, 2048)


def kernel(input_ids, table, gamma, beta):
    bsh = input_ids.shape
    idx = input_ids.reshape(-1).astype(jnp.int32)
    b = idx.shape[0]
    g2 = gamma.reshape(1, HIDDEN)
    b2 = beta.reshape(1, HIDDEN)
    out = None
    row0 = 0
    for sl in SLICES:
        gathered = _gather_sc(table, idx, row0, sl)
        out = _layernorm_tc_slice(gathered, g2, b2, out, row0, b)
        row0 += sl
    return out.reshape(*bsh, HIDDEN)


# speedup vs baseline: 1.0269x; 1.0269x over previous
import functools

import jax
import jax.numpy as jnp
from jax import lax
from jax.experimental import pallas as pl
from jax.experimental.pallas import tpu as pltpu
from jax.experimental.pallas import tpu_sc as plsc

HIDDEN = 1024
EPS = 1e-12

NC = 2
NS = 16
NW = NC * NS

CHUNK = 16
NBUF = 6
PRIME = 4


def _gather_sc(table, idx, row0, b):
    b_per_w = b // NW
    n_chunks = b_per_w // CHUNK
    n_quads = n_chunks // NBUF
    mesh = plsc.VectorSubcoreMesh(core_axis_name="c", subcore_axis_name="s")

    @functools.partial(
        pl.kernel,
        mesh=mesh,
        out_type=jax.ShapeDtypeStruct((b, HIDDEN), jnp.float32),
        scratch_types=[
            pltpu.VMEM((b_per_w,), jnp.int32),
        ] + [pltpu.VMEM((CHUNK, HIDDEN), jnp.float32)] * NBUF
          + [pltpu.SemaphoreType.DMA] * (2 * NBUF),
    )
    def k(table_hbm, idx_hbm, out_hbm, idx_v, *rest):
        bufs = rest[:NBUF]
        gs = rest[NBUF:2 * NBUF]
        ws = rest[2 * NBUF:]
        wid = lax.axis_index("s") * NC + lax.axis_index("c")
        base = wid * b_per_w
        pltpu.sync_copy(idx_hbm.at[pl.ds(row0 + base, b_per_w)], idx_v)

        def start_gather(c, buf, sem):
            pltpu.async_copy(
                table_hbm.at[idx_v.at[pl.ds(c * CHUNK, CHUNK)]], buf, sem
            )

        def wait_gather(buf, sem):
            pltpu.make_async_copy(
                table_hbm.at[idx_v.at[pl.ds(0, CHUNK)]], buf, sem
            ).wait()

        def start_write(c, buf, sem):
            pltpu.async_copy(buf, out_hbm.at[pl.ds(base + c * CHUNK, CHUNK)], sem)

        def wait_write(buf, sem):
            pltpu.make_async_copy(
                buf, out_hbm.at[pl.ds(base, CHUNK)], sem
            ).wait()

        for c0 in range(PRIME):
            start_gather(c0, bufs[c0 % NBUF], gs[c0 % NBUF])

        for c in range(n_chunks):
            bix = c % NBUF
            wait_gather(bufs[bix], gs[bix])
            start_write(c, bufs[bix], ws[bix])
            nxt = c + PRIME
            if nxt < n_chunks:
                pb = nxt % NBUF
                if nxt >= NBUF:
                    wait_write(bufs[pb], ws[pb])
                start_gather(nxt, bufs[pb], gs[pb])

        for bix in range(NBUF):
            if bix < n_chunks:
                wait_write(bufs[bix], ws[bix])

    return k(table, idx)


BT = 2048


def _layernorm_tc_slice(x, gamma, beta, acc, row0, b_total, nrows=None):
    sl = nrows if nrows is not None else x.shape[0]
    off = row0 // BT

    def body(*refs):
        x_ref, g_ref, b_ref, o_ref = refs[-4:]
        v = x_ref[...]
        m = jnp.mean(v, axis=1, keepdims=True)
        c = v - m
        var = jnp.mean(c * c, axis=1, keepdims=True)
        o_ref[...] = c * lax.rsqrt(var + EPS) * g_ref[...] + b_ref[...]

    x_spec = pl.BlockSpec((BT, HIDDEN), lambda i: (i, 0))
    vec_spec = pl.BlockSpec((1, HIDDEN), lambda i: (0, 0))
    if acc is None:
        in_specs = [x_spec, vec_spec, vec_spec]
        args = (x, gamma, beta)
        aliases = {}
    else:
        in_specs = [pl.BlockSpec(memory_space=pl.ANY), x_spec, vec_spec,
                    vec_spec]
        args = (acc, x, gamma, beta)
        aliases = {0: 0}
    return pl.pallas_call(
        body,
        grid=(sl // BT,),
        in_specs=in_specs,
        out_specs=pl.BlockSpec((BT, HIDDEN), lambda i: (off + i, 0)),
        out_shape=jax.ShapeDtypeStruct((b_total, HIDDEN), jnp.float32),
        input_output_aliases=aliases,
    )(*args)


SLICES = (2048, 10240, 4096)


def kernel(input_ids, table, gamma, beta):
    bsh = input_ids.shape
    idx = input_ids.reshape(-1).astype(jnp.int32)
    b = idx.shape[0]
    g2 = gamma.reshape(1, HIDDEN)
    b2 = beta.reshape(1, HIDDEN)
    out = None
    row0 = 0
    for sl in SLICES:
        gathered = _gather_sc(table, idx, row0, sl)
        out = _layernorm_tc_slice(gathered, g2, b2, out, row0, b)
        row0 += sl
    return out.reshape(*bsh, HIDDEN)


# speedup vs baseline: 1.0367x; 1.0096x over previous
import functools

import jax
import jax.numpy as jnp
from jax import lax
from jax.experimental import pallas as pl
from jax.experimental.pallas import tpu as pltpu
from jax.experimental.pallas import tpu_sc as plsc

HIDDEN = 1024
EPS = 1e-12

NC = 2
NS = 16
NW = NC * NS

CHUNK = 16
NBUF = 6
PRIME = 5


def _gather_sc(table, idx, row0, b):
    b_per_w = b // NW
    n_chunks = b_per_w // CHUNK
    n_quads = n_chunks // NBUF
    mesh = plsc.VectorSubcoreMesh(core_axis_name="c", subcore_axis_name="s")

    @functools.partial(
        pl.kernel,
        mesh=mesh,
        out_type=jax.ShapeDtypeStruct((b, HIDDEN), jnp.float32),
        scratch_types=[
            pltpu.VMEM((b_per_w,), jnp.int32),
        ] + [pltpu.VMEM((CHUNK, HIDDEN), jnp.float32)] * NBUF
          + [pltpu.SemaphoreType.DMA] * (2 * NBUF),
    )
    def k(table_hbm, idx_hbm, out_hbm, idx_v, *rest):
        bufs = rest[:NBUF]
        gs = rest[NBUF:2 * NBUF]
        ws = rest[2 * NBUF:]
        wid = lax.axis_index("s") * NC + lax.axis_index("c")
        base = wid * b_per_w
        pltpu.sync_copy(idx_hbm.at[pl.ds(row0 + base, b_per_w)], idx_v)

        def start_gather(c, buf, sem):
            pltpu.async_copy(
                table_hbm.at[idx_v.at[pl.ds(c * CHUNK, CHUNK)]], buf, sem
            )

        def wait_gather(buf, sem):
            pltpu.make_async_copy(
                table_hbm.at[idx_v.at[pl.ds(0, CHUNK)]], buf, sem
            ).wait()

        def start_write(c, buf, sem):
            pltpu.async_copy(buf, out_hbm.at[pl.ds(base + c * CHUNK, CHUNK)], sem)

        def wait_write(buf, sem):
            pltpu.make_async_copy(
                buf, out_hbm.at[pl.ds(base, CHUNK)], sem
            ).wait()

        for c0 in range(PRIME):
            start_gather(c0, bufs[c0 % NBUF], gs[c0 % NBUF])

        for c in range(n_chunks):
            bix = c % NBUF
            wait_gather(bufs[bix], gs[bix])
            start_write(c, bufs[bix], ws[bix])
            nxt = c + PRIME
            if nxt < n_chunks:
                pb = nxt % NBUF
                if nxt >= NBUF:
                    wait_write(bufs[pb], ws[pb])
                start_gather(nxt, bufs[pb], gs[pb])

        for bix in range(NBUF):
            if bix < n_chunks:
                wait_write(bufs[bix], ws[bix])

    return k(table, idx)


BT = 2048


def _layernorm_tc_slice(x, gamma, beta, acc, row0, b_total, nrows=None):
    sl = nrows if nrows is not None else x.shape[0]
    off = row0 // BT

    def body(*refs):
        x_ref, g_ref, b_ref, o_ref = refs[-4:]
        v = x_ref[...]
        m = jnp.mean(v, axis=1, keepdims=True)
        c = v - m
        var = jnp.mean(c * c, axis=1, keepdims=True)
        o_ref[...] = c * lax.rsqrt(var + EPS) * g_ref[...] + b_ref[...]

    x_spec = pl.BlockSpec((BT, HIDDEN), lambda i: (i, 0))
    vec_spec = pl.BlockSpec((1, HIDDEN), lambda i: (0, 0))
    if acc is None:
        in_specs = [x_spec, vec_spec, vec_spec]
        args = (x, gamma, beta)
        aliases = {}
    else:
        in_specs = [pl.BlockSpec(memory_space=pl.ANY), x_spec, vec_spec,
                    vec_spec]
        args = (acc, x, gamma, beta)
        aliases = {0: 0}
    return pl.pallas_call(
        body,
        grid=(sl // BT,),
        in_specs=in_specs,
        out_specs=pl.BlockSpec((BT, HIDDEN), lambda i: (off + i, 0)),
        out_shape=jax.ShapeDtypeStruct((b_total, HIDDEN), jnp.float32),
        input_output_aliases=aliases,
    )(*args)


SLICES = (4096, 8192, 4096)


def kernel(input_ids, table, gamma, beta):
    bsh = input_ids.shape
    idx = input_ids.reshape(-1).astype(jnp.int32)
    b = idx.shape[0]
    g2 = gamma.reshape(1, HIDDEN)
    b2 = beta.reshape(1, HIDDEN)
    out = None
    row0 = 0
    for sl in SLICES:
        gathered = _gather_sc(table, idx, row0, sl)
        out = _layernorm_tc_slice(gathered, g2, b2, out, row0, b)
        row0 += sl
    return out.reshape(*bsh, HIDDEN)
